# baseline (speedup 1.0000x reference)
import functools

import jax
import jax.numpy as jnp
from jax import lax
from jax.experimental import pallas as pl
from jax.experimental.pallas import tpu as pltpu
from jax.experimental.pallas import tpu_sc as plsc

_TOPK = 8


def _router_body(x_ref, wrt_ref, br_ref, xb_ref, logits_ref):
  xb = x_ref[...].astype(jnp.bfloat16)
  xb_ref[...] = xb
  logits_ref[...] = jnp.dot(xb, wrt_ref[...].astype(jnp.bfloat16),
                            preferred_element_type=jnp.float32) + br_ref[...]


def _sc_routing_body(rows_w, E, logits_hbm, probs_hbm, pacc_hbm, sacc_hbm,
                     lvm, pvm, app, asel, sem):
  info = plsc.get_sparse_core_info()
  nc = info.num_cores
  wid = lax.axis_index("s") * nc + lax.axis_index("c")
  base = wid * rows_w * E

  pltpu.sync_copy(logits_hbm.at[pl.ds(base, rows_w * E)], lvm)
  app[...] = jnp.zeros((E,), jnp.float32)
  asel[...] = jnp.zeros((E,), jnp.float32)

  lane = lax.iota(jnp.int32, E)

  def one_row(i):
    lv = lvm[pl.ds(i * E, E)]
    ks, _ = plsc.sort_key_val(lv, lane)
    v8 = jnp.sum(jnp.where(lane == E - _TOPK, ks, 0.0))
    mx = jnp.sum(jnp.where(lane == E - 1, ks, 0.0))
    gt = lv > v8
    eq = lv == v8
    n_gt = jnp.sum(gt.astype(jnp.int32))
    ceq = plsc.cumsum(eq.astype(jnp.int32))
    sel = jnp.logical_or(gt, jnp.logical_and(eq, ceq <= (_TOPK - n_gt)))
    eqm = lv == mx
    c1 = plsc.cumsum(eqm.astype(jnp.int32))
    oh1 = jnp.logical_and(eqm, c1 == 1)
    ex = jnp.where(sel, jnp.exp(lv - mx), 0.0)
    p = ex / jnp.sum(ex)
    pvm[pl.ds(i * E, E)] = p
    return p, oh1

  def row2(i, carry):
    pa, oa = one_row(2 * i)
    pb, ob = one_row(2 * i + 1)
    app[...] += pa + pb
    asel[...] += oa.astype(jnp.float32) + ob.astype(jnp.float32)
    return carry

  lax.fori_loop(0, rows_w // 2, row2, 0)

  pltpu.sync_copy(pvm, probs_hbm.at[pl.ds(base, rows_w * E)])
  pltpu.sync_copy(app, pacc_hbm.at[pl.ds(wid * E, E)])
  pltpu.sync_copy(asel, sacc_hbm.at[pl.ds(wid * E, E)])


def _moe_body(eb, T, E, p_ref, x_ref, we_ref, be_ref, pacc_ref, sacc_ref,
              out_ref, fp_ref, lbl_ref):
  e4 = pl.program_id(2)
  p = p_ref[...]
  x = x_ref[...]
  iot = jax.lax.broadcasted_iota(jnp.int32, p.shape, 1)

  contrib = None
  for j in range(eb):
    acc = jnp.dot(x, we_ref[j],
                  preferred_element_type=jnp.float32)
    pcol = jnp.sum(jnp.where(iot == e4 * eb + j, p, 0.0),
                   axis=1, keepdims=True)
    term = acc * pcol
    contrib = term if contrib is None else contrib + term

  @pl.when(e4 == 0)
  def _init():
    out_ref[...] = contrib + jnp.dot(
        p.astype(jnp.bfloat16), be_ref[...].astype(jnp.bfloat16),
        preferred_element_type=jnp.float32)

  @pl.when(e4 != 0)
  def _acc():
    out_ref[...] += contrib

  t = pl.program_id(0)
  h = pl.program_id(1)
  last = ((t == pl.num_programs(0) - 1) & (h == pl.num_programs(1) - 1)
          & (e4 == pl.num_programs(2) - 1))

  @pl.when(last)
  def _stats():
    fp = jnp.sum(pacc_ref[...], axis=0, keepdims=True) / jnp.float32(T)
    fs = jnp.sum(sacc_ref[...], axis=0, keepdims=True) / jnp.float32(T)
    fp_ref[...] = fp
    lbl_ref[...] = jnp.full((1, 1), jnp.float32(E)) * jnp.sum(
        fs * fp, keepdims=True)


def kernel(x, Wr, br, We, be):
  T, H_IN = x.shape
  E = Wr.shape[0]
  H_OUT = We.shape[2]

  bt_r = min(1024, T)
  router = pl.pallas_call(
      _router_body,
      grid=(T // bt_r,),
      in_specs=[
          pl.BlockSpec((bt_r, H_IN), lambda i: (i, 0)),
          pl.BlockSpec((H_IN, E), lambda i: (0, 0)),
          pl.BlockSpec((1, E), lambda i: (0, 0)),
      ],
      out_specs=[
          pl.BlockSpec((bt_r, H_IN), lambda i: (i, 0)),
          pl.BlockSpec((bt_r, E), lambda i: (i, 0)),
      ],
      out_shape=[
          jax.ShapeDtypeStruct((T, H_IN), jnp.bfloat16),
          jax.ShapeDtypeStruct((T, E), jnp.float32),
      ],
      compiler_params=pltpu.CompilerParams(
          dimension_semantics=("parallel",)),
  )
  xb, logits = router(x, Wr.T, br.reshape(1, E))

  info = plsc.get_sparse_core_info()
  nw = info.num_cores * info.num_subcores
  rows_w = T // nw
  mesh = plsc.VectorSubcoreMesh(core_axis_name="c", subcore_axis_name="s")
  sc_routing = pl.kernel(
      functools.partial(_sc_routing_body, rows_w, E),
      mesh=mesh,
      out_type=[
          jax.ShapeDtypeStruct((T * E,), jnp.float32),
          jax.ShapeDtypeStruct((nw * E,), jnp.float32),
          jax.ShapeDtypeStruct((nw * E,), jnp.float32),
      ],
      scratch_types=[
          pltpu.VMEM((rows_w * E,), jnp.float32),
          pltpu.VMEM((rows_w * E,), jnp.float32),
          pltpu.VMEM((E,), jnp.float32),
          pltpu.VMEM((E,), jnp.float32),
          pltpu.SemaphoreType.DMA,
      ],
      compiler_params=pltpu.CompilerParams(needs_layout_passes=False),
  )
  probs_flat, pacc, sacc = sc_routing(logits.reshape(T * E))
  probs = probs_flat.reshape(T, E)
  pacc = pacc.reshape(nw, E)
  sacc = sacc.reshape(nw, E)

  bt = min(2048, T)
  bh = min(256, H_OUT)
  eb = min(8, E)
  web = We.astype(jnp.bfloat16)
  moe = pl.pallas_call(
      functools.partial(_moe_body, eb, T, E),
      grid=(T // bt, H_OUT // bh, E // eb),
      in_specs=[
          pl.BlockSpec((bt, E), lambda t, h, e: (t, 0)),
          pl.BlockSpec((bt, H_IN), lambda t, h, e: (t, 0)),
          pl.BlockSpec((eb, H_IN, bh), lambda t, h, e: (e, 0, h)),
          pl.BlockSpec((E, bh), lambda t, h, e: (0, h)),
          pl.BlockSpec((nw, E), lambda t, h, e: (0, 0)),
          pl.BlockSpec((nw, E), lambda t, h, e: (0, 0)),
      ],
      out_specs=[
          pl.BlockSpec((bt, bh), lambda t, h, e: (t, h)),
          pl.BlockSpec((1, E), lambda t, h, e: (0, 0)),
          pl.BlockSpec((1, 1), lambda t, h, e: (0, 0)),
      ],
      out_shape=[
          jax.ShapeDtypeStruct((T, H_OUT), jnp.float32),
          jax.ShapeDtypeStruct((1, E), jnp.float32),
          jax.ShapeDtypeStruct((1, 1), jnp.float32),
      ],
      compiler_params=pltpu.CompilerParams(
          dimension_semantics=("parallel", "parallel", "arbitrary")),
  )
  out, fp2, lbl2 = moe(probs, xb, web, be, pacc, sacc)

  return (out, fp2.reshape(E, 1), lbl2.reshape(()))


# speedup vs baseline: 1.0677x; 1.0677x over previous
import functools

import jax
import jax.numpy as jnp
from jax import lax
from jax.experimental import pallas as pl
from jax.experimental.pallas import tpu as pltpu
from jax.experimental.pallas import tpu_sc as plsc

_TOPK = 8


def _router_body(x_ref, wrt_ref, br_ref, xb_ref, logits_ref):
  xb = x_ref[...].astype(jnp.bfloat16)
  xb_ref[...] = xb
  logits_ref[...] = jnp.dot(xb, wrt_ref[...].astype(jnp.bfloat16),
                            preferred_element_type=jnp.float32) + br_ref[...]


def _sc_routing_body(rows_w, E, logits_hbm, probs_hbm, pacc_hbm, sacc_hbm,
                     lvm, pvm, app, asel, sem):
  info = plsc.get_sparse_core_info()
  nc = info.num_cores
  wid = lax.axis_index("s") * nc + lax.axis_index("c")
  base = wid * rows_w * E

  pltpu.sync_copy(logits_hbm.at[pl.ds(base, rows_w * E)], lvm)
  app[...] = jnp.zeros((E,), jnp.float32)
  asel[...] = jnp.zeros((E,), jnp.float32)

  lane = lax.iota(jnp.int32, E)

  def one_row(i):
    lv = lvm[pl.ds(i * E, E)]
    ks, _ = plsc.sort_key_val(lv, lane)
    v8 = jnp.sum(jnp.where(lane == E - _TOPK, ks, 0.0))
    mx = jnp.sum(jnp.where(lane == E - 1, ks, 0.0))
    gt = lv > v8
    eq = lv == v8
    n_gt = jnp.sum(gt.astype(jnp.int32))
    ceq = plsc.cumsum(eq.astype(jnp.int32))
    sel = jnp.logical_or(gt, jnp.logical_and(eq, ceq <= (_TOPK - n_gt)))
    eqm = lv == mx
    c1 = plsc.cumsum(eqm.astype(jnp.int32))
    oh1 = jnp.logical_and(eqm, c1 == 1)
    ex = jnp.where(sel, jnp.exp(lv - mx), 0.0)
    p = ex / jnp.sum(ex)
    pvm[pl.ds(i * E, E)] = p
    return p, oh1

  def row2(i, carry):
    pa, oa = one_row(2 * i)
    pb, ob = one_row(2 * i + 1)
    app[...] += pa + pb
    asel[...] += oa.astype(jnp.float32) + ob.astype(jnp.float32)
    return carry

  lax.fori_loop(0, rows_w // 2, row2, 0)

  pltpu.sync_copy(pvm, probs_hbm.at[pl.ds(base, rows_w * E)])
  pltpu.sync_copy(app, pacc_hbm.at[pl.ds(wid * E, E)])
  pltpu.sync_copy(asel, sacc_hbm.at[pl.ds(wid * E, E)])


def _moe_body(eb, T, E, p_ref, x_ref, we_ref, be_ref, pacc_ref, sacc_ref,
              out_ref, fp_ref, lbl_ref):
  e4 = pl.program_id(2)
  p = p_ref[...]
  x = x_ref[...]
  iot = jax.lax.broadcasted_iota(jnp.int32, p.shape, 1)

  contrib = None
  for j in range(eb):
    acc = jnp.dot(x, we_ref[j].astype(jnp.bfloat16),
                  preferred_element_type=jnp.float32)
    pcol = jnp.sum(jnp.where(iot == e4 * eb + j, p, 0.0),
                   axis=1, keepdims=True)
    term = acc * pcol
    contrib = term if contrib is None else contrib + term

  @pl.when(e4 == 0)
  def _init():
    out_ref[...] = contrib + jnp.dot(
        p.astype(jnp.bfloat16), be_ref[...].astype(jnp.bfloat16),
        preferred_element_type=jnp.float32)

  @pl.when(e4 != 0)
  def _acc():
    out_ref[...] += contrib

  t = pl.program_id(0)
  h = pl.program_id(1)
  last = ((t == pl.num_programs(0) - 1) & (h == pl.num_programs(1) - 1)
          & (e4 == pl.num_programs(2) - 1))

  @pl.when(last)
  def _stats():
    fp = jnp.sum(pacc_ref[...], axis=0, keepdims=True) / jnp.float32(T)
    fs = jnp.sum(sacc_ref[...], axis=0, keepdims=True) / jnp.float32(T)
    fp_ref[...] = fp
    lbl_ref[...] = jnp.full((1, 1), jnp.float32(E)) * jnp.sum(
        fs * fp, keepdims=True)


def kernel(x, Wr, br, We, be):
  T, H_IN = x.shape
  E = Wr.shape[0]
  H_OUT = We.shape[2]

  bt_r = min(1024, T)
  router = pl.pallas_call(
      _router_body,
      grid=(T // bt_r,),
      in_specs=[
          pl.BlockSpec((bt_r, H_IN), lambda i: (i, 0)),
          pl.BlockSpec((H_IN, E), lambda i: (0, 0)),
          pl.BlockSpec((1, E), lambda i: (0, 0)),
      ],
      out_specs=[
          pl.BlockSpec((bt_r, H_IN), lambda i: (i, 0)),
          pl.BlockSpec((bt_r, E), lambda i: (i, 0)),
      ],
      out_shape=[
          jax.ShapeDtypeStruct((T, H_IN), jnp.bfloat16),
          jax.ShapeDtypeStruct((T, E), jnp.float32),
      ],
      compiler_params=pltpu.CompilerParams(
          dimension_semantics=("parallel",)),
  )
  xb, logits = router(x, Wr.T, br.reshape(1, E))

  info = plsc.get_sparse_core_info()
  nw = info.num_cores * info.num_subcores
  rows_w = T // nw
  mesh = plsc.VectorSubcoreMesh(core_axis_name="c", subcore_axis_name="s")
  sc_routing = pl.kernel(
      functools.partial(_sc_routing_body, rows_w, E),
      mesh=mesh,
      out_type=[
          jax.ShapeDtypeStruct((T * E,), jnp.float32),
          jax.ShapeDtypeStruct((nw * E,), jnp.float32),
          jax.ShapeDtypeStruct((nw * E,), jnp.float32),
      ],
      scratch_types=[
          pltpu.VMEM((rows_w * E,), jnp.float32),
          pltpu.VMEM((rows_w * E,), jnp.float32),
          pltpu.VMEM((E,), jnp.float32),
          pltpu.VMEM((E,), jnp.float32),
          pltpu.SemaphoreType.DMA,
      ],
      compiler_params=pltpu.CompilerParams(needs_layout_passes=False),
  )
  probs_flat, pacc, sacc = sc_routing(logits.reshape(T * E))
  probs = probs_flat.reshape(T, E)
  pacc = pacc.reshape(nw, E)
  sacc = sacc.reshape(nw, E)

  bt = min(2048, T)
  bh = min(256, H_OUT)
  eb = min(4, E)
  moe = pl.pallas_call(
      functools.partial(_moe_body, eb, T, E),
      grid=(T // bt, H_OUT // bh, E // eb),
      in_specs=[
          pl.BlockSpec((bt, E), lambda t, h, e: (t, 0)),
          pl.BlockSpec((bt, H_IN), lambda t, h, e: (t, 0)),
          pl.BlockSpec((eb, H_IN, bh), lambda t, h, e: (e, 0, h)),
          pl.BlockSpec((E, bh), lambda t, h, e: (0, h)),
          pl.BlockSpec((nw, E), lambda t, h, e: (0, 0)),
          pl.BlockSpec((nw, E), lambda t, h, e: (0, 0)),
      ],
      out_specs=[
          pl.BlockSpec((bt, bh), lambda t, h, e: (t, h)),
          pl.BlockSpec((1, E), lambda t, h, e: (0, 0)),
          pl.BlockSpec((1, 1), lambda t, h, e: (0, 0)),
      ],
      out_shape=[
          jax.ShapeDtypeStruct((T, H_OUT), jnp.float32),
          jax.ShapeDtypeStruct((1, E), jnp.float32),
          jax.ShapeDtypeStruct((1, 1), jnp.float32),
      ],
      compiler_params=pltpu.CompilerParams(
          dimension_semantics=("parallel", "parallel", "arbitrary")),
  )
  out, fp2, lbl2 = moe(probs, xb, We, be, pacc, sacc)

  return (out, fp2.reshape(E, 1), lbl2.reshape(()))
